# baseline (speedup 1.0000x reference)
import jax
import jax.numpy as jnp
from jax.experimental import pallas as pl
from jax.experimental.pallas import tpu as pltpu

BATCH = 4096
DIM = 512
NUM_CLASSES = 10000
MARGIN = 0.35

B_TILE = 512
C_TILE = 2048
NB = BATCH // B_TILE
NC = -(-NUM_CLASSES // C_TILE)
LAST_VALID = NUM_CLASSES - (NC - 1) * C_TILE

LANES = 128
NCHUNK = C_TILE // LANES


def _rownorm(x):
    return x / (jnp.sqrt(jnp.sum(x * x, axis=1, keepdims=True)) + 1e-12)


def _body(f_ref, c_ref, lbl_ref, out_ref, fn_scr, cn_scr, s_scr, t_scr):
    j = pl.program_id(0)
    i = pl.program_id(1)

    @pl.when(j == 0)
    def _():
        fn_scr[i] = _rownorm(f_ref[...])

    @pl.when(i == 0)
    def _():
        cn_scr[...] = _rownorm(c_ref[...])

    logits = jax.lax.dot_general(
        fn_scr[i], cn_scr[...], (((1,), (1,)), ((), ())),
        preferred_element_type=jnp.float32,
    )

    lbl = lbl_ref[0, 0, :]
    lane = jax.lax.broadcasted_iota(jnp.int32, (B_TILE, LANES), 1)

    def fused_sums(valid_upto):
        s_acc = None
        t_acc = None
        for k in range(NCHUNK):
            if k * LANES >= valid_upto:
                break
            lg = logits[:, k * LANES:(k + 1) * LANES]
            e_k = jnp.exp(lg)
            if (k + 1) * LANES > valid_upto:
                e_k = jnp.where(lane < valid_upto - k * LANES, e_k, 0.0)
            m_k = jnp.where(
                lane == lbl[:, None] - (j * C_TILE + k * LANES), lg, 0.0
            )
            s_acc = e_k if s_acc is None else s_acc + e_k
            t_acc = m_k if t_acc is None else t_acc + m_k
        return s_acc, t_acc

    @pl.when(j == 0)
    def _():
        s_acc, t_acc = fused_sums(C_TILE)
        s_scr[i] = s_acc
        t_scr[i] = t_acc

    @pl.when(jnp.logical_and(j > 0, j < NC - 1))
    def _():
        s_acc, t_acc = fused_sums(C_TILE)
        s_scr[i] = s_scr[i] + s_acc
        t_scr[i] = t_scr[i] + t_acc

    @pl.when(j == NC - 1)
    def _():
        s_acc, t_acc = fused_sums(LAST_VALID)
        t = jnp.sum(t_scr[i] + t_acc, axis=1)
        tm = t - MARGIN
        s = jnp.sum(s_scr[i] + s_acc, axis=1) - jnp.exp(t) + jnp.exp(tm)
        out_ref[0, :] = jnp.log(s) - tm


def kernel(feature, label, centers):
    lbl3 = label.reshape(NB, 1, B_TILE)
    out = pl.pallas_call(
        _body,
        grid=(NC, NB),
        in_specs=[
            pl.BlockSpec((B_TILE, DIM), lambda j, i: (i, 0)),
            pl.BlockSpec((C_TILE, DIM), lambda j, i: (j, 0)),
            pl.BlockSpec((1, 1, B_TILE), lambda j, i: (i, 0, 0)),
        ],
        out_specs=pl.BlockSpec((1, B_TILE), lambda j, i: (0, i)),
        out_shape=jax.ShapeDtypeStruct((1, BATCH), jnp.float32),
        scratch_shapes=[
            pltpu.VMEM((NB, B_TILE, DIM), jnp.float32),
            pltpu.VMEM((C_TILE, DIM), jnp.float32),
            pltpu.VMEM((NB, B_TILE, LANES), jnp.float32),
            pltpu.VMEM((NB, B_TILE, LANES), jnp.float32),
        ],
    )(feature, centers, lbl3)
    return out.reshape(BATCH)


# speedup vs baseline: 1.4864x; 1.4864x over previous
import jax
import jax.numpy as jnp
from jax.experimental import pallas as pl
from jax.experimental.pallas import tpu as pltpu

BATCH = 4096
DIM = 512
NUM_CLASSES = 10000
MARGIN = 0.35

B_TILE = 512
C_TILE = 2048
NB = BATCH // B_TILE
NC = -(-NUM_CLASSES // C_TILE)
LAST_VALID = NUM_CLASSES - (NC - 1) * C_TILE

LOG2E = 1.4426950408889634
LN2 = 0.6931471805599453

LANES = 128
NCHUNK = C_TILE // LANES


def _rownorm(x):
    return x / (jnp.sqrt(jnp.sum(x * x, axis=1, keepdims=True)) + 1e-12)


def _chunk_sum(x):
    acc = x[:, :LANES]
    for k in range(1, NCHUNK):
        acc = acc + x[:, k * LANES:(k + 1) * LANES]
    return acc


def _body(f_ref, c_ref, lbl_ref, out_ref, fn_scr, cn_scr, s_scr, t_scr):
    j = pl.program_id(0)
    i = pl.program_id(1)

    @pl.when(j == 0)
    def _():
        fn_scr[i] = _rownorm(f_ref[...]) * LOG2E

    @pl.when(i == 0)
    def _():
        cn_scr[...] = _rownorm(c_ref[...])

    logits = jax.lax.dot_general(
        fn_scr[i], cn_scr[...], (((1,), (1,)), ((), ())),
        preferred_element_type=jnp.float32,
    )

    e = jnp.exp2(logits)
    lbl = lbl_ref[0, 0, :]
    cols = j * C_TILE + jax.lax.broadcasted_iota(jnp.int32, (B_TILE, C_TILE), 1)
    masked = jnp.where(cols == lbl[:, None], logits, 0.0)
    t_part = _chunk_sum(masked)

    @pl.when(j == 0)
    def _():
        s_scr[i] = _chunk_sum(e)
        t_scr[i] = t_part

    @pl.when(jnp.logical_and(j > 0, j < NC - 1))
    def _():
        s_scr[i] = s_scr[i] + _chunk_sum(e)
        t_scr[i] = t_scr[i] + t_part

    @pl.when(j == NC - 1)
    def _():
        lane = jax.lax.broadcasted_iota(jnp.int32, (B_TILE, C_TILE), 1)
        ee = jnp.where(lane < LAST_VALID, e, 0.0)
        s128 = s_scr[i] + _chunk_sum(ee)
        t = jnp.sum(t_scr[i] + t_part, axis=1) * LN2
        tm = t - MARGIN
        s = jnp.sum(s128, axis=1) - jnp.exp(t) + jnp.exp(tm)
        out_ref[0, :] = jnp.log(s) - tm


def kernel(feature, label, centers):
    lbl3 = label.reshape(NB, 1, B_TILE)
    out = pl.pallas_call(
        _body,
        grid=(NC, NB),
        in_specs=[
            pl.BlockSpec((B_TILE, DIM), lambda j, i: (i, 0)),
            pl.BlockSpec((C_TILE, DIM), lambda j, i: (j, 0)),
            pl.BlockSpec((1, 1, B_TILE), lambda j, i: (i, 0, 0)),
        ],
        out_specs=pl.BlockSpec((1, B_TILE), lambda j, i: (0, i)),
        out_shape=jax.ShapeDtypeStruct((1, BATCH), jnp.float32),
        scratch_shapes=[
            pltpu.VMEM((NB, B_TILE, DIM), jnp.float32),
            pltpu.VMEM((C_TILE, DIM), jnp.float32),
            pltpu.VMEM((NB, B_TILE, LANES), jnp.float32),
            pltpu.VMEM((NB, B_TILE, LANES), jnp.float32),
        ],
    )(feature, centers, lbl3)
    return out.reshape(BATCH)
